# baseline (speedup 1.0000x reference)
import jax
import jax.numpy as jnp
from jax.experimental import pallas as pl
from jax.experimental.pallas import tpu as pltpu

_NEG = -100000000.0
_BB = 2


def _entropy_body(x_ref, out_ref):
    x = x_ref[...]
    g = x * jnp.log2(jnp.maximum(x, 1e-37))
    k = jax.lax.broadcasted_iota(jnp.int32, (1, 1, 1, x.shape[3]), 3)
    gk = jnp.where(k >= 1, g, 0.0)
    out_ref[:, 0, :] = -gk.sum(axis=(1, 3))


def _select_body(ent_ref, x256_ref, cmf_ref, out_mask_ref, out_idx_ref):
    x256 = x256_ref[...]
    g = x256 * jnp.log2(jnp.maximum(x256, 1e-37))
    cmf = cmf_ref[...]
    e = (ent_ref[...] - g.sum(axis=1)) * cmf
    p = None
    for d in (-17, -16, -15, -1, 0, 1, 15, 16, 17):
        s = e if d == 0 else jnp.roll(e, -d, axis=1)
        p = s if p is None else p + s
    p = p / 9.0
    ii = jax.lax.broadcasted_iota(jnp.int32, e.shape, 1)
    r = ii >> 4
    c = ii & 15
    border = (r == 0) | (r == 15) | (c == 0) | (c == 15)
    p = jnp.where(border, _NEG, p)
    m = jnp.max(p, axis=1, keepdims=True)
    cand = jnp.where(p == m, ii, 256)
    idx = jnp.min(cand, axis=1, keepdims=True)
    r0 = idx >> 4
    c0 = idx & 15
    neigh = (jnp.abs(r - r0) <= 1) & (jnp.abs(c - c0) <= 1)
    keep = neigh | (cmf == 0.0)
    out_mask_ref[...] = keep.astype(jnp.float32)
    k = jax.lax.broadcasted_iota(jnp.int32, out_idx_ref.shape, 1)
    off = (k // 3) * 16 + (k % 3)
    out_idx_ref[...] = (idx - 17) + off


def kernel(attn, current_mask, mask_indices, glimpse_num):
    B, H, S, _ = attn.shape
    N = S - 1
    ent = pl.pallas_call(
        _entropy_body,
        grid=(B // _BB,),
        in_specs=[pl.BlockSpec((_BB, H, S, N), lambda b: (b, 0, 0, 0))],
        out_specs=pl.BlockSpec((_BB, 1, S), lambda b: (b, 0, 0)),
        out_shape=jax.ShapeDtypeStruct((B, 1, S), jnp.float32),
        compiler_params=pltpu.CompilerParams(
            dimension_semantics=("parallel",)),
    )(attn)
    ent_s = ent[:, 0, 1:]
    x256 = attn[:, :, 1:, N]
    cmf = jnp.where(current_mask, 0.0, 1.0).astype(jnp.float32)
    maskf, new_idx = pl.pallas_call(
        _select_body,
        out_shape=(
            jax.ShapeDtypeStruct((B, N), jnp.float32),
            jax.ShapeDtypeStruct((B, 9), jnp.int32),
        ),
    )(ent_s, x256, cmf)
    out_mask = maskf > 0.5
    out_idx = jnp.concatenate([mask_indices, new_idx.astype(mask_indices.dtype)],
                              axis=1)
    return (out_mask, out_idx)


# speedup vs baseline: 1.2876x; 1.2876x over previous
import jax
import jax.numpy as jnp
from jax.experimental import pallas as pl
from jax.experimental.pallas import tpu as pltpu

_NEG = -100000000.0
_BB = 2


def _entropy_body(x_ref, out_ref):
    x = x_ref[...]
    g = x * jnp.log2(jnp.maximum(x, 1e-37))
    k = jax.lax.broadcasted_iota(jnp.int32, (1, 1, 1, x.shape[3]), 3)
    gk = jnp.where(k >= 1, g, 0.0)
    out_ref[:, 0, :] = -gk.sum(axis=(1, 3))


def _select_body(ent_ref, cmf_ref, out_mask_ref, out_idx_ref):
    cmf = cmf_ref[...]
    e = ent_ref[...] * cmf
    p = None
    for d in (-17, -16, -15, -1, 0, 1, 15, 16, 17):
        s = e if d == 0 else jnp.roll(e, -d, axis=1)
        p = s if p is None else p + s
    p = p / 9.0
    ii = jax.lax.broadcasted_iota(jnp.int32, e.shape, 1)
    r = ii >> 4
    c = ii & 15
    border = (r == 0) | (r == 15) | (c == 0) | (c == 15)
    p = jnp.where(border, _NEG, p)
    m = jnp.max(p, axis=1, keepdims=True)
    cand = jnp.where(p == m, ii, 256)
    idx = jnp.min(cand, axis=1, keepdims=True)
    r0 = idx >> 4
    c0 = idx & 15
    neigh = (jnp.abs(r - r0) <= 1) & (jnp.abs(c - c0) <= 1)
    keep = neigh | (cmf == 0.0)
    out_mask_ref[...] = keep.astype(jnp.float32)
    k = jax.lax.broadcasted_iota(jnp.int32, out_idx_ref.shape, 1)
    off = (k // 3) * 16 + (k % 3)
    out_idx_ref[...] = (idx - 17) + off


def kernel(attn, current_mask, mask_indices, glimpse_num):
    B, H, S, _ = attn.shape
    N = S - 1
    ent = pl.pallas_call(
        _entropy_body,
        grid=(B // _BB,),
        in_specs=[pl.BlockSpec((_BB, H, S, S), lambda b: (b, 0, 0, 0))],
        out_specs=pl.BlockSpec((_BB, 1, S), lambda b: (b, 0, 0)),
        out_shape=jax.ShapeDtypeStruct((B, 1, S), jnp.float32),
        compiler_params=pltpu.CompilerParams(
            dimension_semantics=("parallel",)),
    )(attn)
    ent_s = ent[:, 0, 1:]
    cmf = jnp.where(current_mask, 0.0, 1.0).astype(jnp.float32)
    maskf, new_idx = pl.pallas_call(
        _select_body,
        out_shape=(
            jax.ShapeDtypeStruct((B, N), jnp.float32),
            jax.ShapeDtypeStruct((B, 9), jnp.int32),
        ),
    )(ent_s, cmf)
    out_mask = maskf > 0.5
    out_idx = jnp.concatenate([mask_indices, new_idx.astype(mask_indices.dtype)],
                              axis=1)
    return (out_mask, out_idx)


# speedup vs baseline: 1.2970x; 1.0073x over previous
import jax
import jax.numpy as jnp
from jax.experimental import pallas as pl
from jax.experimental.pallas import tpu as pltpu

_NEG = -100000000.0
_BB = 4


def _entropy_body(x_ref, out_ref):
    x = x_ref[...]
    g = x * jnp.log2(jnp.maximum(x, 1e-37))
    k = jax.lax.broadcasted_iota(jnp.int32, (1, 1, 1, x.shape[3]), 3)
    gk = jnp.where(k >= 1, g, 0.0)
    out_ref[:, 0, :] = -gk.sum(axis=(1, 3))


def _select_body(ent_ref, cmf_ref, out_mask_ref, out_idx_ref):
    cmf = cmf_ref[...]
    e = ent_ref[...] * cmf
    p = None
    for d in (-17, -16, -15, -1, 0, 1, 15, 16, 17):
        s = e if d == 0 else jnp.roll(e, -d, axis=1)
        p = s if p is None else p + s
    p = p / 9.0
    ii = jax.lax.broadcasted_iota(jnp.int32, e.shape, 1)
    r = ii >> 4
    c = ii & 15
    border = (r == 0) | (r == 15) | (c == 0) | (c == 15)
    p = jnp.where(border, _NEG, p)
    m = jnp.max(p, axis=1, keepdims=True)
    cand = jnp.where(p == m, ii, 256)
    idx = jnp.min(cand, axis=1, keepdims=True)
    r0 = idx >> 4
    c0 = idx & 15
    neigh = (jnp.abs(r - r0) <= 1) & (jnp.abs(c - c0) <= 1)
    keep = neigh | (cmf == 0.0)
    out_mask_ref[...] = keep.astype(jnp.float32)
    k = jax.lax.broadcasted_iota(jnp.int32, out_idx_ref.shape, 1)
    off = (k // 3) * 16 + (k % 3)
    out_idx_ref[...] = (idx - 17) + off


def kernel(attn, current_mask, mask_indices, glimpse_num):
    B, H, S, _ = attn.shape
    N = S - 1
    ent = pl.pallas_call(
        _entropy_body,
        grid=(B // _BB,),
        in_specs=[pl.BlockSpec((_BB, H, S, S), lambda b: (b, 0, 0, 0))],
        out_specs=pl.BlockSpec((_BB, 1, S), lambda b: (b, 0, 0)),
        out_shape=jax.ShapeDtypeStruct((B, 1, S), jnp.float32),
        compiler_params=pltpu.CompilerParams(
            dimension_semantics=("parallel",)),
    )(attn)
    ent_s = ent[:, 0, 1:]
    cmf = jnp.where(current_mask, 0.0, 1.0).astype(jnp.float32)
    maskf, new_idx = pl.pallas_call(
        _select_body,
        out_shape=(
            jax.ShapeDtypeStruct((B, N), jnp.float32),
            jax.ShapeDtypeStruct((B, 9), jnp.int32),
        ),
    )(ent_s, cmf)
    out_mask = maskf > 0.5
    out_idx = jnp.concatenate([mask_indices, new_idx.astype(mask_indices.dtype)],
                              axis=1)
    return (out_mask, out_idx)
